# Initial kernel scaffold; baseline (speedup 1.0000x reference)
import jax
import jax.numpy as jnp
from jax.experimental import pallas as pl


def kernel(x, edge_index, batch, conv1_W1, conv1_b1, conv1_W2, conv1_b2, pool1_w, conv2_W1, conv2_b1, conv2_W2, conv2_b2, pool2_w, conv3_W1, conv3_b1, conv3_W2, conv3_b2, pool3_w, conv4_W1, conv4_b1, conv4_W2, conv4_b2, pool4_w, lin1_W, lin1_b, lin2_W, lin2_b, lin3_W, lin3_b):
    raise NotImplementedError("write your pallas kernel here")


# speedup vs baseline: 8.0752x; 8.0752x over previous
import functools
import math

import jax
import jax.numpy as jnp
import numpy as np
from jax import lax
from jax.experimental import pallas as pl
from jax.experimental.pallas import tpu as pltpu
from jax.experimental.pallas import tpu_sc as plsc

N = 10000
F = 128
NPAD = 10240
NR = NPAD // 128
BLK = 512
G = NPAD // BLK
CHUNK = 128
NTILES = 32
NCH = -(-320000 // (NTILES * CHUNK))
E_PAD = NTILES * NCH * CHUNK
EPT = NCH * CHUNK
RPT = NPAD // 16
INT_MIN = np.int32(-2147483648)
NEG_INF = np.float32(-np.inf)


@functools.lru_cache(maxsize=None)
def _build_agg():
    mesh = plsc.VectorSubcoreMesh(core_axis_name="c", subcore_axis_name="s")
    out = jax.ShapeDtypeStruct((NPAD, F), jnp.float32)

    @functools.partial(
        pl.kernel, mesh=mesh, out_type=[out, out],
        scratch_types=[
            pltpu.VMEM((CHUNK,), jnp.int32),
            pltpu.VMEM((CHUNK,), jnp.int32),
            pltpu.VMEM((CHUNK, F), jnp.float32),
            pltpu.VMEM_SHARED((NPAD, F), jnp.float32),
            pltpu.SemaphoreType.DMA,
        ],
    )
    def agg(h_hbm, src_hbm, dst_hbm, zeros_hbm, out0, out1, sidx, didx, rows,
            acc, sem):
        c = lax.axis_index("c")
        s = lax.axis_index("s")
        r0 = pl.multiple_of(s * RPT, 8)
        pltpu.sync_copy(zeros_hbm.at[pl.ds(r0, RPT)], acc.at[pl.ds(r0, RPT)])
        plsc.subcore_barrier()
        base0 = (c * 16 + s) * EPT

        def body(i, carry):
            base = pl.multiple_of(base0 + i * CHUNK, 8)
            pltpu.sync_copy(src_hbm.at[pl.ds(base, CHUNK)], sidx)
            pltpu.sync_copy(dst_hbm.at[pl.ds(base, CHUNK)], didx)
            pltpu.async_copy(h_hbm.at[sidx], rows, sem).wait()
            pltpu.sync_copy(rows, acc.at[didx], add=True)
            return carry

        lax.fori_loop(0, NCH, body, 0)
        plsc.subcore_barrier()

        @pl.when(c == 0)
        def _():
            pltpu.sync_copy(acc.at[pl.ds(r0, RPT)], out0.at[pl.ds(r0, RPT)])

        @pl.when(c == 1)
        def _():
            pltpu.sync_copy(acc.at[pl.ds(r0, RPT)], out1.at[pl.ds(r0, RPT)])

    return agg


def dense_body(hm_ref, p0_ref, p1_ref, w1_ref, b1_ref, w2_ref, b2_ref,
               wm_ref, nrm_ref, valid_ref, h2_ref, score_ref, key_ref):
    z = hm_ref[...] + p0_ref[...] + p1_ref[...]
    t = jnp.maximum(jnp.dot(z, w1_ref[...]) + b1_ref[...], 0.0)
    h2 = jnp.maximum(jnp.dot(t, w2_ref[...]) + b2_ref[...], 0.0)
    h2_ref[...] = h2
    s = jnp.dot(h2, wm_ref[...])[:, :1]
    score = jnp.tanh(s / nrm_ref[0, 0])
    score_ref[...] = score
    bits = lax.bitcast_convert_type(score, jnp.int32)
    key = jnp.where(bits >= 0, bits, INT_MIN - bits)
    key_ref[...] = jnp.where(valid_ref[...] > 0, key, INT_MIN)


def _dense_call():
    node = lambda: pl.BlockSpec((BLK, F), lambda i: (i, 0))
    mat = lambda: pl.BlockSpec((F, F), lambda i: (0, 0))
    row = lambda: pl.BlockSpec((1, F), lambda i: (0, 0))
    col = lambda: pl.BlockSpec((BLK, 1), lambda i: (i, 0))
    one = lambda: pl.BlockSpec((1, 1), lambda i: (0, 0))
    return pl.pallas_call(
        dense_body,
        grid=(G,),
        in_specs=[node(), node(), node(), mat(), row(), mat(), row(), mat(),
                  one(), col()],
        out_specs=[node(), col(), col()],
        out_shape=[jax.ShapeDtypeStruct((NPAD, F), jnp.float32),
                   jax.ShapeDtypeStruct((NPAD, 1), jnp.float32),
                   jax.ShapeDtypeStruct((NPAD, 1), jnp.int32)],
    )


def _kth_largest(vals, mask, k):
    cnt_nonneg = jnp.sum((mask & (vals >= 0)).astype(jnp.int32))
    pos = cnt_nonneg >= k
    lo = jnp.where(pos, jnp.int32(0), INT_MIN + 2)
    hi = jnp.where(pos, jnp.int32(2147483646), jnp.int32(-1))

    def body(_, lh):
        lo, hi = lh
        mid = lo + ((hi - lo + 1) >> 1)
        go = jnp.sum((mask & (vals >= mid)).astype(jnp.int32)) >= k
        return (jnp.where(go, mid, lo), jnp.where(go, hi, mid - 1))

    lo, hi = lax.fori_loop(0, 31, body, (lo, hi))
    return lo


def select_body(k_keep, n_prev, *refs):
    key_ref = refs[0]
    pk_refs = refs[1:1 + n_prev]
    sel_ref = refs[1 + n_prev]
    key = key_ref[...]
    t1 = _kth_largest(key, key > INT_MIN, k_keep)
    sel = key > t1
    grp = key == t1
    m = k_keep - jnp.sum(sel.astype(jnp.int32))
    for pk_ref in pk_refs:
        pk = pk_ref[...]
        t = _kth_largest(pk, grp, m)
        win = grp & (pk > t)
        sel = sel | win
        m = m - jnp.sum(win.astype(jnp.int32))
        grp = grp & (pk == t)
    idx = (lax.broadcasted_iota(jnp.int32, (NR, 128), 0) * 128
           + lax.broadcasted_iota(jnp.int32, (NR, 128), 1))

    def tbody(_, lh):
        lo, hi = lh
        mid = (lo + hi) >> 1
        go = jnp.sum((grp & (idx <= mid)).astype(jnp.int32)) >= m
        return (jnp.where(go, lo, mid + 1), jnp.where(go, mid, hi))

    jlo, _ = lax.fori_loop(0, 14, tbody, (jnp.int32(0), jnp.int32(NPAD - 1)))
    sel_ref[...] = (sel | (grp & (idx <= jlo))).astype(jnp.float32)


def _select_call(k_keep, n_prev):
    return pl.pallas_call(
        functools.partial(select_body, k_keep, n_prev),
        out_shape=jax.ShapeDtypeStruct((NR, 128), jnp.float32),
    )


def pool_body(k_keep, h2_ref, score_ref, selm_ref, reads_ref, hmn_ref,
              rout_ref):
    i = pl.program_id(0)
    sm = selm_ref[...]
    hv = h2_ref[...] * (score_ref[...] * sm)
    hmn_ref[...] = hv
    bmax = jnp.max(jnp.where(sm > 0, hv, NEG_INF), axis=0)
    bsum = jnp.sum(hv, axis=0)

    @pl.when(i == 0)
    def _():
        rout_ref[...] = jnp.concatenate([bmax, bsum])[None, :]

    @pl.when(i > 0)
    def _():
        prev = rout_ref[...]
        pm = jnp.maximum(prev[:, :F], bmax[None, :])
        ps = prev[:, F:] + bsum[None, :]
        rout_ref[...] = jnp.concatenate([pm, ps], axis=1)

    @pl.when(i == G - 1)
    def _():
        prev = rout_ref[...]
        rout_ref[...] = reads_ref[...] + jnp.concatenate(
            [prev[:, :F], prev[:, F:] / k_keep], axis=1)


def _pool_call(k_keep):
    node = lambda: pl.BlockSpec((BLK, F), lambda i: (i, 0))
    col = lambda: pl.BlockSpec((BLK, 1), lambda i: (i, 0))
    rd = lambda: pl.BlockSpec((1, 2 * F), lambda i: (0, 0))
    return pl.pallas_call(
        functools.partial(pool_body, k_keep),
        grid=(G,),
        in_specs=[node(), col(), col(), rd()],
        out_specs=[node(), rd()],
        out_shape=[jax.ShapeDtypeStruct((NPAD, F), jnp.float32),
                   jax.ShapeDtypeStruct((1, 2 * F), jnp.float32)],
    )


def head_body(reads_ref, l1w_ref, l1b_ref, l2w_ref, l2b_ref, l3w_ref, l3b_ref,
              ls_ref, sg_ref):
    z = jnp.maximum(jnp.dot(reads_ref[...], l1w_ref[...]) + l1b_ref[...], 0.0)
    z = jnp.maximum(jnp.dot(z, l2w_ref[...]) + l2b_ref[...], 0.0)
    logits = jnp.dot(z, l3w_ref[...]) + l3b_ref[...]
    mx = jnp.max(logits, axis=1, keepdims=True)
    lse = jnp.log(jnp.sum(jnp.exp(logits - mx), axis=1, keepdims=True)) + mx
    ls_ref[...] = logits - lse
    sg_ref[...] = 1.0 / (1.0 + jnp.exp(-logits))


def _head_call():
    out = jax.ShapeDtypeStruct((1, 10), jnp.float32)
    return pl.pallas_call(head_body, out_shape=[out, out])


def kernel(x, edge_index, batch, conv1_W1, conv1_b1, conv1_W2, conv1_b2,
           pool1_w, conv2_W1, conv2_b1, conv2_W2, conv2_b2, pool2_w,
           conv3_W1, conv3_b1, conv3_W2, conv3_b2, pool3_w,
           conv4_W1, conv4_b1, conv4_W2, conv4_b2, pool4_w,
           lin1_W, lin1_b, lin2_W, lin2_b, lin3_W, lin3_b):
    src = edge_index[0]
    dst = edge_index[1]
    order = jnp.argsort(dst, stable=True)
    epad = jnp.full((E_PAD - src.shape[0],), NPAD - 1, jnp.int32)
    src_p = jnp.concatenate([src[order], epad])
    dst_p = jnp.concatenate([dst[order], epad])

    hm = jnp.zeros((NPAD, F), jnp.float32).at[:N].set(x)
    valid = jnp.zeros((NPAD, 1), jnp.float32).at[:N].set(1.0)
    reads = jnp.zeros((1, 2 * F), jnp.float32)
    zeros = jnp.zeros((NPAD, F), jnp.float32)

    convs = [(conv1_W1, conv1_b1, conv1_W2, conv1_b2),
             (conv2_W1, conv2_b1, conv2_W2, conv2_b2),
             (conv3_W1, conv3_b1, conv3_W2, conv3_b2),
             (conv4_W1, conv4_b1, conv4_W2, conv4_b2)]
    pools = [pool1_w, pool2_w, pool3_w, pool4_w]

    n_cur = N
    pks = []
    for (w1, b1, w2, b2), pw in zip(convs, pools):
        k_keep = int(math.ceil(0.75 * n_cur))
        p0, p1 = _build_agg()(hm, src_p, dst_p, zeros)
        wmat = jnp.zeros((F, F), jnp.float32).at[:, 0].set(pw)
        nrm = jnp.sqrt(jnp.sum(pw * pw)).reshape(1, 1)
        h2, score, key = _dense_call()(
            hm, p0, p1, w1, b1.reshape(1, F), w2, b2.reshape(1, F),
            wmat, nrm, valid)
        key2d = key.reshape(NR, 128)
        sel2d = _select_call(k_keep, len(pks))(key2d, *pks)
        valid = sel2d.reshape(NPAD, 1)
        hm, reads = _pool_call(k_keep)(h2, score, valid, reads)
        pks.insert(0, key2d)
        n_cur = k_keep

    ls, sg = _head_call()(
        reads, lin1_W, lin1_b.reshape(1, -1), lin2_W, lin2_b.reshape(1, -1),
        lin3_W, lin3_b.reshape(1, -1))
    return ls, sg
